# baseline (speedup 1.0000x reference)
import jax
import jax.numpy as jnp
from jax.experimental import pallas as pl

E = 16
G = 4
EPG = E // G
LG = 2
K = 2
BLK = 512


def _router_kernel(w_ref, x_ref, wout_ref, iout_ref):
    w = w_ref[...]
    x = x_ref[...]
    scores = jax.lax.dot_general(
        w, x, (((1,), (1,)), ((), ())),
        preferred_element_type=jnp.float32)
    m = jnp.max(scores, axis=0, keepdims=True)
    e = jnp.exp(scores - m)
    p = e / jnp.sum(e, axis=0, keepdims=True)
    gs = jnp.concatenate(
        [jnp.max(p[g * EPG:(g + 1) * EPG], axis=0, keepdims=True)
         for g in range(G)], axis=0)
    rowg = jax.lax.broadcasted_iota(jnp.int32, gs.shape, 0)
    g1v = jnp.max(gs, axis=0, keepdims=True)
    g1 = jnp.min(jnp.where(gs == g1v, rowg, G), axis=0, keepdims=True)
    gs2 = jnp.where(rowg == g1, -jnp.inf, gs)
    g2v = jnp.max(gs2, axis=0, keepdims=True)
    g2 = jnp.min(jnp.where(gs2 == g2v, rowg, G), axis=0, keepdims=True)
    rowe = jax.lax.broadcasted_iota(jnp.int32, p.shape, 0)
    egrp = rowe // EPG
    allowed = (egrp == g1) | (egrp == g2)
    neg = jnp.float32(-jnp.inf)
    sel = jnp.where(allowed, p, neg)
    m1 = jnp.max(sel, axis=0, keepdims=True)
    i1 = jnp.min(jnp.where(sel == m1, rowe, E), axis=0, keepdims=True)
    sel2 = jnp.where(rowe == i1, neg, sel)
    m2 = jnp.max(sel2, axis=0, keepdims=True)
    i2 = jnp.min(jnp.where(sel2 == m2, rowe, E), axis=0, keepdims=True)
    wout_ref[...] = jnp.concatenate([m1, m2], axis=0)
    iout_ref[...] = jnp.concatenate([i1, i2], axis=0)


@jax.jit
def kernel(x, W):
    n, d = x.shape
    grid = (n // BLK,)
    wout, iout = pl.pallas_call(
        _router_kernel,
        grid=grid,
        in_specs=[pl.BlockSpec((E, d), lambda i: (0, 0)),
                  pl.BlockSpec((BLK, d), lambda i: (i, 0))],
        out_specs=[pl.BlockSpec((K, BLK), lambda i: (0, i)),
                   pl.BlockSpec((K, BLK), lambda i: (0, i))],
        out_shape=[jax.ShapeDtypeStruct((K, n), jnp.float32),
                   jax.ShapeDtypeStruct((K, n), jnp.int32)],
    )(W, x)
    return wout.T, iout.T


# speedup vs baseline: 1.2300x; 1.2300x over previous
import jax
import jax.numpy as jnp
from jax.experimental import pallas as pl
from jax.experimental.pallas import tpu as pltpu

E = 16
G = 4
EPG = E // G
LG = 2
K = 2
BLK = 1024


def _router_kernel(w_ref, x_ref, wout_ref, iout_ref):
    w = w_ref[...]
    x = x_ref[...]
    scores = jax.lax.dot_general(
        w, x, (((1,), (1,)), ((), ())),
        preferred_element_type=jnp.float32)
    m = jnp.max(scores, axis=0, keepdims=True)
    e = jnp.exp(scores - m)
    p = e / jnp.sum(e, axis=0, keepdims=True)
    gs = jnp.concatenate(
        [jnp.max(p[g * EPG:(g + 1) * EPG], axis=0, keepdims=True)
         for g in range(G)], axis=0)
    rowg = jax.lax.broadcasted_iota(jnp.int32, gs.shape, 0)
    g1v = jnp.max(gs, axis=0, keepdims=True)
    g1 = jnp.min(jnp.where(gs == g1v, rowg, G), axis=0, keepdims=True)
    gs2 = jnp.where(rowg == g1, -jnp.inf, gs)
    g2v = jnp.max(gs2, axis=0, keepdims=True)
    g2 = jnp.min(jnp.where(gs2 == g2v, rowg, G), axis=0, keepdims=True)
    rowe = jax.lax.broadcasted_iota(jnp.int32, p.shape, 0)
    egrp = rowe // EPG
    allowed = (egrp == g1) | (egrp == g2)
    neg = jnp.float32(-jnp.inf)
    sel = jnp.where(allowed, p, neg)
    m1 = jnp.max(sel, axis=0, keepdims=True)
    i1 = jnp.min(jnp.where(sel == m1, rowe, E), axis=0, keepdims=True)
    sel2 = jnp.where(rowe == i1, neg, sel)
    m2 = jnp.max(sel2, axis=0, keepdims=True)
    i2 = jnp.min(jnp.where(sel2 == m2, rowe, E), axis=0, keepdims=True)
    wout_ref[...] = jnp.concatenate([m1, m2], axis=0)
    iout_ref[...] = jnp.concatenate([i1, i2], axis=0)


@jax.jit
def kernel(x, W):
    n, d = x.shape
    grid = (n // BLK,)
    wout, iout = pl.pallas_call(
        _router_kernel,
        grid=grid,
        in_specs=[pl.BlockSpec((E, d), lambda i: (0, 0)),
                  pl.BlockSpec((BLK, d), lambda i: (i, 0))],
        out_specs=[pl.BlockSpec((K, BLK), lambda i: (0, i)),
                   pl.BlockSpec((K, BLK), lambda i: (0, i))],
        out_shape=[jax.ShapeDtypeStruct((K, n), jnp.float32),
                   jax.ShapeDtypeStruct((K, n), jnp.int32)],
        compiler_params=pltpu.CompilerParams(
            dimension_semantics=("parallel",)),
    )(W, x)
    return wout.T, iout.T
